# Initial kernel scaffold; baseline (speedup 1.0000x reference)
import jax
import jax.numpy as jnp
from jax.experimental import pallas as pl


def kernel(features, src, rel, dst, w1, bias1, w2, bias2):
    raise NotImplementedError("write your pallas kernel here")


# speedup vs baseline: 22.3932x; 22.3932x over previous
import functools

import jax
import jax.numpy as jnp
from jax import lax
from jax.experimental import pallas as pl
from jax.experimental.pallas import tpu as pltpu
from jax.experimental.pallas import tpu_sc as plsc

N = 10000
F_IN = 128
EMB = 16
NCLS = 16
R_RAW = 2
NR = 2 * R_RAW + 1
E_RAW = 320000
TBL = N * NR

NC = 2
NS = 16
NT = NC * NS
LANES = 16

BATCH = 128
KB = 160
EPT = KB * BATCH
EP = EPT * NT
ROWS_PER_SUB = TBL // NS
TBL_P = 50176
CNT_PER_SUB = TBL_P // NS
PAD_ROW = NR - 1

_mesh = plsc.VectorSubcoreMesh(
    core_axis_name="c", subcore_axis_name="s", num_cores=NC, num_subcores=NS
)


@functools.partial(
    pl.kernel,
    out_type=jax.ShapeDtypeStruct((NT, TBL_P), jnp.float32),
    mesh=_mesh,
    scratch_types=[
        pltpu.VMEM((EPT,), jnp.int32),
        pltpu.VMEM((TBL_P,), jnp.float32),
    ],
    compiler_params=pltpu.CompilerParams(needs_layout_passes=False),
)
def _count_kernel(tidx_hbm, zeros_hbm, out_hbm, idx_v, hist_v):
    c = lax.axis_index("c")
    s = lax.axis_index("s")
    w = c * NS + s
    pltpu.sync_copy(tidx_hbm.at[pl.ds(w * EPT, EPT)], idx_v)
    pltpu.sync_copy(zeros_hbm, hist_v)
    ones = jnp.full((LANES,), 1.0, jnp.float32)

    @pl.loop(0, EPT // LANES)
    def _(i):
        v = idx_v[pl.ds(i * LANES, LANES)]
        plsc.addupdate_scatter(hist_v, [v], ones)

    pltpu.sync_copy(hist_v, out_hbm.at[w])


@functools.partial(
    pl.kernel,
    out_type=jax.ShapeDtypeStruct((NC, NS, ROWS_PER_SUB, EMB), jnp.float32),
    mesh=_mesh,
    scratch_types=[
        pltpu.VMEM((KB, BATCH), jnp.int32),
        pltpu.VMEM((KB, BATCH), jnp.int32),
        pltpu.VMEM((BATCH, EMB), jnp.float32),
        pltpu.VMEM((BATCH, EMB), jnp.float32),
        pltpu.VMEM_SHARED((TBL, EMB), jnp.float32),
        pltpu.SemaphoreType.DMA,
        pltpu.SemaphoreType.DMA,
    ],
    compiler_params=pltpu.CompilerParams(use_tc_tiling_on_sc=False),
)
def _edge_pass_kernel(y_hbm, g_hbm, t_hbm, z_hbm, out_hbm,
                      gi_v, ti_v, rows0, rows1, acc_sh, sem0, sem1):
    c = lax.axis_index("c")
    s = lax.axis_index("s")
    w = c * NS + s
    pltpu.sync_copy(g_hbm.at[pl.ds(w * KB, KB)], gi_v)
    pltpu.sync_copy(t_hbm.at[pl.ds(w * KB, KB)], ti_v)
    pltpu.sync_copy(z_hbm, acc_sh.at[pl.ds(s * ROWS_PER_SUB, ROWS_PER_SUB)])
    plsc.subcore_barrier()

    bufs = (rows0, rows1)
    sems = (sem0, sem1)
    nbuf = 2
    for b in range(nbuf):
        pltpu.async_copy(y_hbm.at[gi_v.at[b]], bufs[b], sems[b])

    @pl.loop(0, KB, step=nbuf)
    def _(j):
        for b in range(nbuf):
            jj = j + b
            pltpu.make_async_copy(y_hbm.at[gi_v.at[jj]], bufs[b], sems[b]).wait()
            pltpu.sync_copy(bufs[b], acc_sh.at[ti_v.at[jj]], add=True)

            @pl.when(jj + nbuf < KB)
            def _():
                pltpu.async_copy(y_hbm.at[gi_v.at[jj + nbuf]], bufs[b], sems[b])

    plsc.subcore_barrier()
    sl = pl.ds(s * ROWS_PER_SUB, ROWS_PER_SUB)
    pltpu.sync_copy(acc_sh.at[sl], out_hbm.at[c, s])


def _inv_body(cnt_ref, inv_ref):
    c = jnp.sum(cnt_ref[...], axis=0)
    inv_ref[...] = jnp.where(c > 0.0, 1.0 / c, 0.0)


def _dense1_body(x_ref, w_ref, y_ref):
    x = x_ref[...]
    for r in range(NR):
        y_ref[:, r, :] = jnp.dot(x, w_ref[r], preferred_element_type=jnp.float32)


def _trans1_body(acc_ref, inv_ref, y1_ref, w2_ref, b1_ref, y2_ref):
    msg = y1_ref[:, NR - 1, :] + b1_ref[0]
    for r in range(NR - 1):
        a = acc_ref[0, :, r, :] + acc_ref[1, :, r, :]
        msg = msg + a * inv_ref[:, r][:, None]
    h = jnp.maximum(msg, 0.0)
    for r in range(NR):
        y2_ref[:, r, :] = jnp.dot(h, w2_ref[r], preferred_element_type=jnp.float32)


def _trans2_body(acc_ref, inv_ref, y2_ref, b2_ref, out_ref):
    msg = y2_ref[:, NR - 1, :] + b2_ref[0]
    for r in range(NR - 1):
        a = acc_ref[0, :, r, :] + acc_ref[1, :, r, :]
        msg = msg + a * inv_ref[:, r][:, None]
    out_ref[...] = msg


def kernel(features, src, rel, dst, w1, bias1, w2, bias2):
    f32 = jnp.float32
    pad = EP - 2 * E_RAW
    gidx = jnp.concatenate([
        dst * NR + rel, src * NR + (rel + R_RAW),
        jnp.full((pad,), PAD_ROW, jnp.int32),
    ])
    tidx = jnp.concatenate([
        src * NR + rel, dst * NR + (rel + R_RAW),
        jnp.full((pad,), PAD_ROW, jnp.int32),
    ])
    gidx2 = gidx.reshape(EP // BATCH, BATCH)
    tidx2 = tidx.reshape(EP // BATCH, BATCH)
    zeros2 = jnp.zeros((ROWS_PER_SUB, EMB), f32)
    zeros1 = jnp.zeros((TBL_P,), f32)

    counts = _count_kernel(tidx, zeros1)

    inv = pl.pallas_call(
        _inv_body,
        out_shape=jax.ShapeDtypeStruct((TBL_P,), f32),
    )(counts)
    inv = inv[:TBL].reshape(N, NR)

    nb = 1000
    y1 = pl.pallas_call(
        _dense1_body,
        grid=(N // nb,),
        in_specs=[
            pl.BlockSpec((nb, F_IN), lambda i: (i, 0)),
            pl.BlockSpec((NR, F_IN, EMB), lambda i: (0, 0, 0)),
        ],
        out_specs=pl.BlockSpec((nb, NR, EMB), lambda i: (i, 0, 0)),
        out_shape=jax.ShapeDtypeStruct((N, NR, EMB), f32),
    )(features, w1)

    acc1 = _edge_pass_kernel(y1.reshape(TBL, EMB), gidx2, tidx2, zeros2)

    y2 = pl.pallas_call(
        _trans1_body,
        grid=(N // nb,),
        in_specs=[
            pl.BlockSpec((NC, nb, NR, EMB), lambda i: (0, i, 0, 0)),
            pl.BlockSpec((nb, NR), lambda i: (i, 0)),
            pl.BlockSpec((nb, NR, EMB), lambda i: (i, 0, 0)),
            pl.BlockSpec((NR, EMB, NCLS), lambda i: (0, 0, 0)),
            pl.BlockSpec((1, EMB), lambda i: (0, 0)),
        ],
        out_specs=pl.BlockSpec((nb, NR, NCLS), lambda i: (i, 0, 0)),
        out_shape=jax.ShapeDtypeStruct((N, NR, NCLS), f32),
    )(acc1.reshape(NC, N, NR, EMB), inv, y1,
      w2, bias1.reshape(1, EMB))

    acc2 = _edge_pass_kernel(y2.reshape(TBL, NCLS), gidx2, tidx2, zeros2)

    out = pl.pallas_call(
        _trans2_body,
        grid=(N // nb,),
        in_specs=[
            pl.BlockSpec((NC, nb, NR, NCLS), lambda i: (0, i, 0, 0)),
            pl.BlockSpec((nb, NR), lambda i: (i, 0)),
            pl.BlockSpec((nb, NR, NCLS), lambda i: (i, 0, 0)),
            pl.BlockSpec((1, NCLS), lambda i: (0, 0)),
        ],
        out_specs=pl.BlockSpec((nb, NCLS), lambda i: (i, 0)),
        out_shape=jax.ShapeDtypeStruct((N, NCLS), f32),
    )(acc2.reshape(NC, N, NR, NCLS), inv, y2, bias2.reshape(1, NCLS))

    return out
